# Initial kernel scaffold; baseline (speedup 1.0000x reference)
import jax
import jax.numpy as jnp
from jax.experimental import pallas as pl


def kernel(x, sample_map, weight, bias):
    raise NotImplementedError("write your pallas kernel here")


# speedup vs baseline: 21.9855x; 21.9855x over previous
import functools

import jax
import jax.numpy as jnp
from jax import lax
from jax.experimental import pallas as pl
from jax.experimental.pallas import tpu as pltpu
from jax.experimental.pallas import tpu_sc as plsc

C = 96
H = 224
W = 224
HW = H * W
K = 9
P = H * W
PK = P * K
NW = 32
CPW = PK // NW
NB = 112
NBLK = CPW // NB
LANES = 16
C_OUT = 96
KC = K * C


def _sc_bilinear_gather(table, cx, cy):
    mesh = plsc.VectorSubcoreMesh(core_axis_name="c", subcore_axis_name="s")

    @functools.partial(
        pl.kernel,
        mesh=mesh,
        compiler_params=pltpu.CompilerParams(use_tc_tiling_on_sc=False),
        out_type=jax.ShapeDtypeStruct((PK, C), jnp.float32),
        scratch_types=[
            pltpu.VMEM((NB,), jnp.float32),
            pltpu.VMEM((NB,), jnp.float32),
            pltpu.VMEM((4, NB), jnp.float32),
            pltpu.VMEM((NB,), jnp.int32),
            pltpu.VMEM((NB,), jnp.int32),
            pltpu.VMEM((NB,), jnp.int32),
            pltpu.VMEM((NB,), jnp.int32),
            pltpu.VMEM((NB, C), jnp.float32),
            pltpu.VMEM((NB, C), jnp.float32),
            pltpu.VMEM((NB, C), jnp.float32),
            pltpu.VMEM((NB, C), jnp.float32),
            pltpu.VMEM((NB, C), jnp.float32),
            pltpu.SemaphoreType.DMA,
        ],
    )
    def sc_kernel(table_h, cx_h, cy_h, s_h,
                  cxv, cyv, wv, i0, i1, i2, i3, r0, r1, r2, r3, sv, sem):
        cid = lax.axis_index("c")
        sid = lax.axis_index("s")
        wid = sid * 2 + cid
        base = wid * CPW

        def blk_body(b, _):
            off = base + b * NB
            pltpu.sync_copy(cx_h.at[pl.ds(off, NB)], cxv)
            pltpu.sync_copy(cy_h.at[pl.ds(off, NB)], cyv)
            for g in range(NB // LANES):
                sl = pl.ds(g * LANES, LANES)
                cxg = cxv[sl]
                cyg = cyv[sl]
                x0 = cxg.astype(jnp.int32)
                y0 = cyg.astype(jnp.int32)
                fx = cxg - x0.astype(jnp.float32)
                fy = cyg - y0.astype(jnp.float32)
                gx = 1.0 - fx
                gy = 1.0 - fy
                x1 = x0 + 1
                y1 = y0 + 1
                vx1 = jnp.where(x1 < W, 1.0, 0.0)
                vy1 = jnp.where(y1 < H, 1.0, 0.0)
                x1c = jnp.minimum(x1, W - 1)
                y1c = jnp.minimum(y1, H - 1)
                wv[0, sl] = gx * gy
                wv[1, sl] = fx * gy * vx1
                wv[2, sl] = gx * fy * vy1
                wv[3, sl] = fx * fy * vx1 * vy1
                base00 = y0 * W
                base01 = y1c * W
                i0[sl] = base00 + x0
                i1[sl] = base00 + x1c
                i2[sl] = base01 + x0
                i3[sl] = base01 + x1c
            cp0 = pltpu.async_copy(table_h.at[i0], r0, sem)
            cp1 = pltpu.async_copy(table_h.at[i1], r1, sem)
            cp2 = pltpu.async_copy(table_h.at[i2], r2, sem)
            cp3 = pltpu.async_copy(table_h.at[i3], r3, sem)
            cp0.wait()
            cp1.wait()
            cp2.wait()
            cp3.wait()

            def grp(g, _):
                gsl = pl.ds(g * LANES, LANES)
                w0v = wv[0, gsl]
                w1v = wv[1, gsl]
                w2v = wv[2, gsl]
                w3v = wv[3, gsl]
                for j in range(LANES):
                    i = g * LANES + j
                    w0 = w0v[j]
                    w1 = w1v[j]
                    w2 = w2v[j]
                    w3 = w3v[j]
                    for c in range(C // LANES):
                        slc = pl.ds(c * LANES, LANES)
                        acc = (r0[i, slc] * w0 + r1[i, slc] * w1
                               + r2[i, slc] * w2 + r3[i, slc] * w3)
                        sv[i, slc] = acc
                return 0

            lax.fori_loop(0, NB // LANES, grp, 0)
            pltpu.sync_copy(sv, s_h.at[pl.ds(off, NB)])
            return 0

        lax.fori_loop(0, NBLK, blk_body, 0)

    return sc_kernel(table, cx, cy)


def _tc_contract(s2, w2, bias2):
    PB = 1024

    def body(s_ref, w_ref, b_ref, o_ref):
        o = lax.dot_general(w_ref[...], s_ref[...],
                            (((1,), (1,)), ((), ())),
                            preferred_element_type=jnp.float32)
        o_ref[...] = o + b_ref[...]

    return pl.pallas_call(
        body,
        grid=(P // PB,),
        in_specs=[
            pl.BlockSpec((PB, KC), lambda i: (i, 0)),
            pl.BlockSpec((C_OUT, KC), lambda i: (0, 0)),
            pl.BlockSpec((C_OUT, 1), lambda i: (0, 0)),
        ],
        out_specs=pl.BlockSpec((C_OUT, PB), lambda i: (0, i)),
        out_shape=jax.ShapeDtypeStruct((C_OUT, P), jnp.float32),
    )(s2, w2, bias2)


def kernel(x, sample_map, weight, bias):
    table = jnp.transpose(x[0], (1, 2, 0)).reshape(HW, C)
    cx = sample_map[..., 0].reshape(PK)
    cy = sample_map[..., 1].reshape(PK)
    s = _sc_bilinear_gather(table, cx, cy)
    s2 = s.reshape(P, KC)
    w2 = jnp.transpose(weight, (0, 2, 1)).reshape(C_OUT, KC)
    out = _tc_contract(s2, w2, bias.reshape(C_OUT, 1))
    return out.reshape(1, C_OUT, H, W)


# speedup vs baseline: 31.3307x; 1.4251x over previous
import functools

import jax
import jax.numpy as jnp
from jax import lax
from jax.experimental import pallas as pl
from jax.experimental.pallas import tpu as pltpu
from jax.experimental.pallas import tpu_sc as plsc

C = 96
H = 224
W = 224
HW = H * W
K = 9
P = H * W
PK = P * K
NW = 32
CPW = PK // NW
NB = 112
NBLK = CPW // NB
LANES = 16
C_OUT = 96
KC = K * C


def _sc_bilinear_gather(table, cx, cy):
    mesh = plsc.VectorSubcoreMesh(core_axis_name="c", subcore_axis_name="s")

    @functools.partial(
        pl.kernel,
        mesh=mesh,
        compiler_params=pltpu.CompilerParams(use_tc_tiling_on_sc=False),
        out_type=jax.ShapeDtypeStruct((PK, C), jnp.float32),
        scratch_types=[
            pltpu.VMEM((2, NB), jnp.float32),
            pltpu.VMEM((2, NB), jnp.float32),
            pltpu.VMEM((2, 4, NB), jnp.float32),
            pltpu.VMEM((2, NB), jnp.int32),
            pltpu.VMEM((2, NB), jnp.int32),
            pltpu.VMEM((2, NB), jnp.int32),
            pltpu.VMEM((2, NB), jnp.int32),
            pltpu.VMEM((2, NB, C), jnp.float32),
            pltpu.VMEM((2, NB, C), jnp.float32),
            pltpu.VMEM((2, NB, C), jnp.float32),
            pltpu.VMEM((2, NB, C), jnp.float32),
            pltpu.VMEM((2, NB, C), jnp.float32),
            pltpu.SemaphoreType.DMA,
            pltpu.SemaphoreType.DMA,
            pltpu.SemaphoreType.DMA,
            pltpu.SemaphoreType.DMA,
            pltpu.SemaphoreType.DMA,
            pltpu.SemaphoreType.DMA,
        ],
    )
    def sc_kernel(table_h, cx_h, cy_h, s_h,
                  cxv, cyv, wv, i0, i1, i2, i3, r0, r1, r2, r3, sv,
                  sem_g0, sem_g1, sem_s0, sem_s1, sem_c0, sem_c1):
        cid = lax.axis_index("c")
        sid = lax.axis_index("s")
        wid = sid * 2 + cid
        base = wid * CPW
        sem_g = (sem_g0, sem_g1)
        sem_s = (sem_s0, sem_s1)
        sem_c = (sem_c0, sem_c1)

        def blk_off(b):
            return base + jnp.minimum(b, NBLK - 1) * NB

        def fire_cxy(b, st):
            off = blk_off(b)
            pltpu.async_copy(cx_h.at[pl.ds(off, NB)], cxv.at[st], sem_c[st])
            pltpu.async_copy(cy_h.at[pl.ds(off, NB)], cyv.at[st], sem_c[st])

        def wait_cxy(st):
            pltpu.make_async_copy(cx_h.at[pl.ds(0, NB)], cxv.at[st],
                                  sem_c[st]).wait()
            pltpu.make_async_copy(cy_h.at[pl.ds(0, NB)], cyv.at[st],
                                  sem_c[st]).wait()

        def compute_idx(st):
            for g in range(NB // LANES):
                sl = pl.ds(g * LANES, LANES)
                cxg = cxv[st, sl]
                cyg = cyv[st, sl]
                x0 = cxg.astype(jnp.int32)
                y0 = cyg.astype(jnp.int32)
                fx = cxg - x0.astype(jnp.float32)
                fy = cyg - y0.astype(jnp.float32)
                gx = 1.0 - fx
                gy = 1.0 - fy
                x1 = x0 + 1
                y1 = y0 + 1
                vx1 = jnp.where(x1 < W, 1.0, 0.0)
                vy1 = jnp.where(y1 < H, 1.0, 0.0)
                x1c = jnp.minimum(x1, W - 1)
                y1c = jnp.minimum(y1, H - 1)
                wv[st, 0, sl] = gx * gy
                wv[st, 1, sl] = fx * gy * vx1
                wv[st, 2, sl] = gx * fy * vy1
                wv[st, 3, sl] = fx * fy * vx1 * vy1
                base00 = y0 * W
                base01 = y1c * W
                i0[st, sl] = base00 + x0
                i1[st, sl] = base00 + x1c
                i2[st, sl] = base01 + x0
                i3[st, sl] = base01 + x1c

        def fire_gathers(st):
            pltpu.async_copy(table_h.at[i0.at[st]], r0.at[st], sem_g[st])
            pltpu.async_copy(table_h.at[i1.at[st]], r1.at[st], sem_g[st])
            pltpu.async_copy(table_h.at[i2.at[st]], r2.at[st], sem_g[st])
            pltpu.async_copy(table_h.at[i3.at[st]], r3.at[st], sem_g[st])

        def wait_gathers(st):
            pltpu.make_async_copy(table_h.at[i0.at[st]], r0.at[st],
                                  sem_g[st]).wait()
            pltpu.make_async_copy(table_h.at[i1.at[st]], r1.at[st],
                                  sem_g[st]).wait()
            pltpu.make_async_copy(table_h.at[i2.at[st]], r2.at[st],
                                  sem_g[st]).wait()
            pltpu.make_async_copy(table_h.at[i3.at[st]], r3.at[st],
                                  sem_g[st]).wait()

        def weighted_sum(st):
            def grp(g, _):
                gsl = pl.ds(g * LANES, LANES)
                w0v = wv[st, 0, gsl]
                w1v = wv[st, 1, gsl]
                w2v = wv[st, 2, gsl]
                w3v = wv[st, 3, gsl]
                for j in range(LANES):
                    i = g * LANES + j
                    w0 = w0v[j]
                    w1 = w1v[j]
                    w2 = w2v[j]
                    w3 = w3v[j]
                    for c in range(C // LANES):
                        slc = pl.ds(c * LANES, LANES)
                        acc = (r0[st, i, slc] * w0 + r1[st, i, slc] * w1
                               + r2[st, i, slc] * w2 + r3[st, i, slc] * w3)
                        sv[st, i, slc] = acc
                return 0

            lax.fori_loop(0, NB // LANES, grp, 0)

        def fire_store(b, st):
            off = blk_off(b)
            pltpu.async_copy(sv.at[st], s_h.at[pl.ds(off, NB)], sem_s[st])

        def wait_store(st):
            pltpu.make_async_copy(sv.at[st], s_h.at[pl.ds(base, NB)],
                                  sem_s[st]).wait()

        pltpu.sync_copy(cx_h.at[pl.ds(base, NB)], cxv.at[0])
        pltpu.sync_copy(cy_h.at[pl.ds(base, NB)], cyv.at[0])
        compute_idx(0)
        fire_gathers(0)
        fire_cxy(1, 1)
        fire_cxy(2, 0)

        def pair_body(t, _):
            b0 = 2 * t
            wait_cxy(1)
            compute_idx(1)
            fire_gathers(1)
            fire_cxy(2 * t + 3, 1)
            wait_gathers(0)

            @pl.when(t > 0)
            def _():
                wait_store(0)

            weighted_sum(0)
            fire_store(b0, 0)
            wait_cxy(0)
            compute_idx(0)
            fire_gathers(0)
            fire_cxy(2 * t + 4, 0)
            wait_gathers(1)

            @pl.when(t > 0)
            def _():
                wait_store(1)

            weighted_sum(1)
            fire_store(b0 + 1, 1)
            return 0

        lax.fori_loop(0, NBLK // 2, pair_body, 0)
        wait_gathers(0)
        wait_store(0)
        wait_store(1)
        wait_cxy(0)
        wait_cxy(1)

    return sc_kernel(table, cx, cy)


def _tc_transpose(x2):
    BLK = 2048

    def body(x_ref, o_ref):
        o_ref[...] = x_ref[...].T

    return pl.pallas_call(
        body,
        grid=(HW // BLK,),
        in_specs=[pl.BlockSpec((C, BLK), lambda i: (0, i))],
        out_specs=pl.BlockSpec((BLK, C), lambda i: (i, 0)),
        out_shape=jax.ShapeDtypeStruct((HW, C), jnp.float32),
    )(x2)


def _tc_contract(s2, w2, bias2):
    PB = 1024

    def body(s_ref, w_ref, b_ref, o_ref):
        o = lax.dot_general(w_ref[...], s_ref[...],
                            (((1,), (1,)), ((), ())),
                            preferred_element_type=jnp.float32)
        o_ref[...] = o + b_ref[...]

    return pl.pallas_call(
        body,
        grid=(P // PB,),
        in_specs=[
            pl.BlockSpec((PB, KC), lambda i: (i, 0)),
            pl.BlockSpec((C_OUT, KC), lambda i: (0, 0)),
            pl.BlockSpec((C_OUT, 1), lambda i: (0, 0)),
        ],
        out_specs=pl.BlockSpec((C_OUT, PB), lambda i: (0, i)),
        out_shape=jax.ShapeDtypeStruct((C_OUT, P), jnp.float32),
    )(s2, w2, bias2)


def kernel(x, sample_map, weight, bias):
    table = _tc_transpose(x.reshape(C, HW))
    cx = sample_map[..., 0].reshape(PK)
    cy = sample_map[..., 1].reshape(PK)
    s = _sc_bilinear_gather(table, cx, cy)
    s2 = s.reshape(P, KC)
    w2 = jnp.transpose(weight, (0, 2, 1)).reshape(C_OUT, KC)
    out = _tc_contract(s2, w2, bias.reshape(C_OUT, 1))
    return out.reshape(1, C_OUT, H, W)
